# baseline (speedup 1.0000x reference)
import functools

import jax
import jax.numpy as jnp
from jax import lax
from jax.experimental import pallas as pl
from jax.experimental.pallas import tpu as pltpu
from jax.experimental.pallas import tpu_sc as plsc

EMBED_DIM = 64
CHUNK = 128
NB = 5
K = 3


@functools.lru_cache(maxsize=None)
def _make_gather(B, D):
    info = plsc.get_sparse_core_info()
    NC, NS = info.num_cores, info.num_subcores
    NW = NC * NS
    assert B % (NW * NB * CHUNK) == 0
    b_per_w = B // NW
    n_chunks = b_per_w // CHUNK
    n_outer = n_chunks // NB
    mesh = plsc.VectorSubcoreMesh(core_axis_name="c", subcore_axis_name="s")

    @functools.partial(
        pl.kernel,
        mesh=mesh,
        out_type=jax.ShapeDtypeStruct((B, D), jnp.float32),
        scratch_types=[
            pltpu.VMEM((b_per_w,), jnp.int32),
            pltpu.VMEM((NB, CHUNK, D), jnp.float32),
            pltpu.SemaphoreType.DMA((NB,)),
            pltpu.SemaphoreType.DMA((NB,)),
        ],
        compiler_params=pltpu.CompilerParams(use_tc_tiling_on_sc=False),
    )
    def gather_kernel(idx_hbm, table_hbm, out_hbm, idx_v, bufs, gsem, wsem):
        wid = lax.axis_index("s") * NC + lax.axis_index("c")
        base = wid * b_per_w
        pltpu.sync_copy(idx_hbm.at[pl.ds(base, b_per_w)], idx_v)

        def start_gather(c, b):
            pltpu.async_copy(
                table_hbm.at[idx_v.at[pl.ds(c * CHUNK, CHUNK)]],
                bufs.at[b],
                gsem.at[b],
            )

        def wait_gather(b):
            pltpu.make_async_copy(
                table_hbm.at[idx_v.at[pl.ds(0, CHUNK)]], bufs.at[b], gsem.at[b]
            ).wait()

        def start_write(c, b):
            pltpu.async_copy(
                bufs.at[b], out_hbm.at[pl.ds(base + c * CHUNK, CHUNK)], wsem.at[b]
            )

        def wait_write(b):
            pltpu.make_async_copy(
                bufs.at[b], out_hbm.at[pl.ds(base, CHUNK)], wsem.at[b]
            ).wait()

        for b in range(K):
            start_gather(b, b)

        def outer(o, carry):
            for b in range(NB):
                c = o * NB + b
                wait_gather(b)
                start_write(c, b)
                nb = (b + K) % NB
                cn = c + K
                if b + K >= NB:
                    @pl.when(cn < n_chunks)
                    def _():
                        wait_write(nb)
                        start_gather(cn, nb)

                else:
                    @pl.when(cn < n_chunks)
                    def _():
                        @pl.when(o > 0)
                        def _():
                            wait_write(nb)

                        start_gather(cn, nb)

            return carry

        lax.fori_loop(0, n_outer, outer, 0)
        for b in range(NB):
            wait_write(b)

    return gather_kernel


def kernel(input, emb):
    Bt, H = input.shape
    D = emb.shape[1]
    idx = input.reshape(Bt * H).astype(jnp.int32)
    out = _make_gather(Bt * H, D)(idx, emb)
    return out.reshape(Bt, H, D)
